# baseline (speedup 1.0000x reference)
import functools

import jax
import jax.numpy as jnp
from jax.experimental import pallas as pl
from jax.experimental.pallas import tpu as pltpu

_BLK = 256
_NBUF = 4


def _elu(x):
    return jnp.where(x > 0, x, jnp.exp(jnp.minimum(x, 0.0)) - 1.0)


def _mlp_body(x_ref,
              g1_ref, l1_ref, g2_ref, l2_ref, g3_ref, l3_ref,
              g4_ref, l4_ref, g5_ref, l5_ref,
              b1_ref, b2_ref, b3_ref, b4_ref, b5_ref,
              o_ref,
              *scratch):
    stgs = scratch[5:5 + 2 * _NBUF]
    sems = scratch[5 + 2 * _NBUF:]
    w1_s, w2_s, w3_s, w4_s, w5_s = scratch[:5]
    bf = jnp.bfloat16

    @pl.when(pl.program_id(0) == 0)
    def _fuse_weights():
        tasks = []
        for g_ref, l_ref, w_s in ((g1_ref, l1_ref, w1_s),
                                  (g2_ref, l2_ref, w2_s),
                                  (g3_ref, l3_ref, w3_s),
                                  (g4_ref, l4_ref, w4_s),
                                  (g5_ref, l5_ref, w5_s)):
            nrows, ncols = g_ref.shape
            rb = min(_BLK, nrows)
            for blk in range(nrows // rb):
                tasks.append((g_ref, l_ref, w_s, blk, rb, ncols))

        bufs = tuple((stgs[2 * i], stgs[2 * i + 1],
                      sems[2 * i], sems[2 * i + 1]) for i in range(_NBUF))
        pending = {}

        def start(t):
            g_ref, l_ref, _, blk, rb, ncols = tasks[t]
            sg, sl_, smg, sml = bufs[t % _NBUF]
            rs = pl.ds(blk * rb, rb)
            ca = pltpu.make_async_copy(g_ref.at[rs, :],
                                       sg.at[:rb, :ncols], smg)
            cb = pltpu.make_async_copy(l_ref.at[rs, :],
                                       sl_.at[:rb, :ncols], sml)
            ca.start(); cb.start()
            pending[t] = (ca, cb)

        for t in range(min(_NBUF, len(tasks))):
            start(t)
        for t in range(len(tasks)):
            ca, cb = pending.pop(t)
            ca.wait(); cb.wait()
            _, _, w_s, blk, rb, ncols = tasks[t]
            sg, sl_, _, _ = bufs[t % _NBUF]
            w_s[:, pl.ds(blk * rb, rb)] = (sg[:rb, :ncols]
                                           + sl_[:rb, :ncols]).T.astype(bf)
            if t + _NBUF < len(tasks):
                start(t + _NBUF)

    h = jnp.dot(x_ref[...].astype(bf), w1_s[...],
                preferred_element_type=jnp.float32)
    h = _elu(h + b1_ref[...]).astype(bf)
    h = jnp.dot(h, w2_s[...], preferred_element_type=jnp.float32)
    h = _elu(h + b2_ref[...]).astype(bf)
    h = jnp.dot(h, w3_s[...], preferred_element_type=jnp.float32)
    h = _elu(h + b3_ref[...]).astype(bf)
    h = jnp.dot(h, w4_s[...], preferred_element_type=jnp.float32)
    h = _elu(h + b4_ref[...]).astype(bf)
    h = jnp.dot(h, w5_s[...], preferred_element_type=jnp.float32)
    o_ref[...] = h + b5_ref[...]


@functools.partial(jax.jit, static_argnames=("block_rows",))
def _mlp(x, gws, lws, bs, block_rows=400):
    n, din = x.shape
    dh = gws[1].shape[0]
    dout = gws[4].shape[0]
    grid = (pl.cdiv(n, block_rows),)

    def row_spec(d):
        return pl.BlockSpec((block_rows, d), lambda i: (i, 0))

    def const_spec(a):
        return pl.BlockSpec(a.shape, lambda i: (0,) * a.ndim)

    any_spec = pl.BlockSpec(memory_space=pl.ANY)
    w_in_specs = [any_spec] * 10
    b_specs = [const_spec(b) for b in bs]

    scratch_shapes = [
        pltpu.VMEM((din, dh), jnp.bfloat16),
        pltpu.VMEM((dh, dh), jnp.bfloat16),
        pltpu.VMEM((dh, dh), jnp.bfloat16),
        pltpu.VMEM((dh, dh), jnp.bfloat16),
        pltpu.VMEM((dh, dout), jnp.bfloat16),
    ] + [pltpu.VMEM((_BLK, dh), jnp.float32)] * (2 * _NBUF) \
      + [pltpu.SemaphoreType.DMA] * (2 * _NBUF)

    gl_interleaved = []
    for g, l in zip(gws, lws):
        gl_interleaved += [g, l]

    return pl.pallas_call(
        _mlp_body,
        grid=grid,
        in_specs=[row_spec(din)] + w_in_specs + b_specs,
        out_specs=row_spec(dout),
        out_shape=jax.ShapeDtypeStruct((n, dout), jnp.float32),
        scratch_shapes=scratch_shapes,
        compiler_params=pltpu.CompilerParams(
            dimension_semantics=("arbitrary",),
            vmem_limit_bytes=63 * 1024 * 1024,
        ),
    )(x, *gl_interleaved, *bs)


def kernel(x, edge_index, g1_W, g1_b, l1_W, l1_b, g2_W, g2_b, l2_W, l2_b,
           g3_W, g3_b, l3_W, l3_b, g4_W, g4_b, l4_W, l4_b,
           g5_W, g5_b, l5_W, l5_b):
    del edge_index
    gws = [g1_W, g2_W, g3_W, g4_W, g5_W]
    lws = [l1_W, l2_W, l3_W, l4_W, l5_W]
    bs = [(g1_b + l1_b).reshape(1, -1), (g2_b + l2_b).reshape(1, -1),
          (g3_b + l3_b).reshape(1, -1), (g4_b + l4_b).reshape(1, -1),
          (g5_b + l5_b).reshape(1, -1)]
    return _mlp(x, gws, lws, bs)
